# Initial kernel scaffold; baseline (speedup 1.0000x reference)
import jax
import jax.numpy as jnp
from jax.experimental import pallas as pl


def kernel(x, edge_index, batch, x_omic, params):
    raise NotImplementedError("write your pallas kernel here")


# speedup vs baseline: 4.7457x; 4.7457x over previous
import functools

import jax
import jax.numpy as jnp
import numpy as np
from jax import lax
from jax.experimental import pallas as pl
from jax.experimental.pallas import tpu as pltpu
from jax.experimental.pallas import tpu_sc as plsc

_BN = 1.0 / np.sqrt(1.0 + 1e-5)

N_NODES = 10000
NT = 32
NB = 320
NPAD = NT * NB
NE = 640000
CHUNK = NE // NT
LROWS = 188
CELL_ROWS = NT * LROWS + 8
BLK = 1024
NG = 8

_SC_PARAMS = pltpu.CompilerParams(
    use_tc_tiling_on_sc=False, needs_layout_passes=False)


def _gelu(x):
    return 0.5 * x * (1.0 + lax.erf(x * np.float32(0.7071067811865476)))


def _mesh():
    return plsc.VectorSubcoreMesh(core_axis_name="c", subcore_axis_name="s")


def _wid():
    return lax.axis_index("s") * 2 + lax.axis_index("c")


def _bin_edges(edge_index):
    @functools.partial(
        pl.kernel,
        mesh=_mesh(),
        compiler_params=_SC_PARAMS,
        out_type=(
            jax.ShapeDtypeStruct((CELL_ROWS, 128), jnp.int32),
            jax.ShapeDtypeStruct((CELL_ROWS, 128), jnp.int32),
            jax.ShapeDtypeStruct((NT, NT), jnp.int32),
            jax.ShapeDtypeStruct((NT, NT), jnp.int32),
        ),
        scratch_types=[
            pltpu.VMEM((2000,), jnp.int32),
            pltpu.VMEM((2000,), jnp.int32),
            pltpu.VMEM((LROWS, 128), jnp.int32),
            pltpu.VMEM((LROWS, 128), jnp.int32),
            pltpu.VMEM((32,), jnp.int32),
            pltpu.VMEM((32,), jnp.int32),
            pltpu.VMEM((32,), jnp.int32),
            pltpu.VMEM((32,), jnp.int32),
            pltpu.VMEM((8, 128), jnp.int32),
            pltpu.VMEM((8, 128), jnp.int32),
        ],
    )
    def k(ei, csrc, cdl, hlen, hstart, sbuf, dbuf, lsrc, ldloc, histv,
          lbase, len16v, startv, pad_src, pad_dl):
        w = _wid()
        ebase = w * CHUNK
        z16 = jnp.zeros((16,), jnp.int32)
        s16full = jnp.full((16,), NB, jnp.int32)
        histv[pl.ds(0, 16)] = z16
        histv[pl.ds(16, 16)] = z16

        for blk in range(CHUNK // 2000):
            pltpu.sync_copy(ei.at[1, pl.ds(ebase + 2000 * blk, 2000)], dbuf)

            def h_body(g, _):
                d16 = dbuf[pl.ds(16 * g, 16)]
                b16 = (d16 * 26215) >> 23
                cnt, last = plsc.scan_count(b16)
                plsc.addupdate_scatter(histv, [b16], cnt, mask=last)
                return 0

            lax.fori_loop(0, 125, h_body, 0)

        h0 = histv[pl.ds(0, 16)]
        h1 = histv[pl.ds(16, 16)]
        r0 = (h0 + 127) & ~127
        r1 = (h1 + 127) & ~127
        c0 = plsc.cumsum(r0)
        c1 = plsc.cumsum(r1)
        tot0 = jnp.max(c0)
        s0 = c0 - r0
        s1 = c1 - r1 + tot0
        lbase[pl.ds(0, 16)] = s0
        lbase[pl.ds(16, 16)] = s1
        startv[pl.ds(0, 16)] = s0 >> 7
        startv[pl.ds(16, 16)] = s1 >> 7
        len16v[pl.ds(0, 16)] = (h0 + 15) & ~15
        len16v[pl.ds(16, 16)] = (h1 + 15) & ~15

        def sent(r, _):
            for j in range(8):
                lsrc[r, pl.ds(16 * j, 16)] = z16
                ldloc[r, pl.ds(16 * j, 16)] = s16full
            return 0

        lax.fori_loop(0, LROWS, sent, 0)

        for blk in range(CHUNK // 2000):
            pltpu.sync_copy(ei.at[0, pl.ds(ebase + 2000 * blk, 2000)], sbuf)
            pltpu.sync_copy(ei.at[1, pl.ds(ebase + 2000 * blk, 2000)], dbuf)

            def p_body(g, _):
                sv = sbuf[pl.ds(16 * g, 16)]
                d16 = dbuf[pl.ds(16 * g, 16)]
                b16 = (d16 * 26215) >> 23
                dl16 = d16 - b16 * NB
                cnt, last = plsc.scan_count(b16)
                base16 = plsc.load_gather(lbase, [b16])
                pos = base16 + cnt - 1
                plsc.store_scatter(lsrc, [pos >> 7, pos & 127], sv)
                plsc.store_scatter(ldloc, [pos >> 7, pos & 127], dl16)
                plsc.addupdate_scatter(lbase, [b16], cnt, mask=last)
                return 0

            lax.fori_loop(0, 125, p_body, 0)

        pltpu.sync_copy(lsrc, csrc.at[pl.ds(LROWS * w, LROWS)])
        pltpu.sync_copy(ldloc, cdl.at[pl.ds(LROWS * w, LROWS)])
        pltpu.sync_copy(len16v, hlen.at[w])
        pltpu.sync_copy(startv, hstart.at[w])

        def padr(r, _):
            for j in range(8):
                pad_src[r, pl.ds(16 * j, 16)] = z16
                pad_dl[r, pl.ds(16 * j, 16)] = s16full
            return 0

        lax.fori_loop(0, 8, padr, 0)

        @pl.when(w == 0)
        def _():
            pltpu.sync_copy(pad_src, csrc.at[pl.ds(NT * LROWS, 8)])
            pltpu.sync_copy(pad_dl, cdl.at[pl.ds(NT * LROWS, 8)])

    return k(edge_index)


def _seg_max(csrc_a, cdl_a, hlen_a, hstart_a, Bm, Am, D):
    nf = D // 16

    @functools.partial(
        pl.kernel,
        mesh=_mesh(),
        compiler_params=_SC_PARAMS,
        out_type=jax.ShapeDtypeStruct((NPAD, D), jnp.float32),
        scratch_types=[
            pltpu.VMEM((8, 128), jnp.int32),
            pltpu.VMEM((8, 128), jnp.int32),
            pltpu.VMEM((BLK, D), jnp.float32),
            pltpu.VMEM((NB + 8, D), jnp.float32),
            pltpu.VMEM((NB, D), jnp.float32),
            pltpu.VMEM((NT, NT), jnp.int32),
            pltpu.VMEM((NT, NT), jnp.int32),
            pltpu.SemaphoreType.DMA,
            pltpu.SemaphoreType.DMA,
        ],
    )
    def k(csrc, cdl, hlen, hstart, B_, A_, hout, idx2d, dloc2d, rows, tbl,
          abuf, hbuf, sbufr, sem, sem2):
        w = _wid()
        acp = pltpu.async_copy(A_.at[pl.ds(NB * w, NB)], abuf, sem2)
        pltpu.sync_copy(hlen, hbuf)
        pltpu.sync_copy(hstart, sbufr)

        neg = jnp.full((16,), -jnp.inf, jnp.float32)

        def initr(r, _):
            for f in range(nf):
                tbl[r, pl.ds(16 * f, 16)] = neg
            return 0

        lax.fori_loop(0, NB + 8, initr, 0)

        woff = (w >> 4) << 4
        wlane = w & 15
        lanes = lax.iota(jnp.int32, 16)

        def per_t(t, _):
            hv = hbuf[t, pl.ds(woff, 16)]
            len16 = jnp.max(jnp.where(lanes == wlane, hv, 0))
            sv = sbufr[t, pl.ds(woff, 16)]
            srow = jnp.max(jnp.where(lanes == wlane, sv, 0))
            baser = LROWS * t + srow
            nblk = (len16 + (BLK - 1)) >> 10

            def per_blk(kk, _):
                pltpu.sync_copy(csrc.at[pl.ds(baser + 8 * kk, 8)], idx2d)
                pltpu.sync_copy(cdl.at[pl.ds(baser + 8 * kk, 8)], dloc2d)
                cps = [
                    pltpu.async_copy(B_.at[idx2d.at[j]],
                                     rows.at[pl.ds(128 * j, 128)], sem)
                    for j in range(8)
                ]
                for cp in cps:
                    cp.wait()
                rem = len16 - (kk << 10)
                ngr = jnp.minimum(rem, BLK) >> 4

                def per_g(g, _):
                    dv = dloc2d[g >> 3, pl.ds((g & 7) * 16, 16)]
                    for j in range(16):
                        d = dv[j]
                        e = g * 16 + j
                        for f in range(nf):
                            sl = pl.ds(16 * f, 16)
                            tbl[d, sl] = jnp.maximum(tbl[d, sl], rows[e, sl])
                    return 0

                lax.fori_loop(0, ngr, per_g, 0)
                return 0

            lax.fori_loop(0, nblk, per_blk, 0)
            return 0

        lax.fori_loop(0, NT, per_t, 0)
        acp.wait()

        zeros = jnp.zeros((16,), jnp.float32)

        def finr(r, _):
            for f in range(nf):
                sl = pl.ds(16 * f, 16)
                m = tbl[r, sl]
                abuf[r, sl] = jnp.where(m > -3.0e38, abuf[r, sl] + m, zeros)
            return 0

        lax.fori_loop(0, NB, finr, 0)
        pltpu.sync_copy(abuf, hout.at[pl.ds(NB * w, NB)])

    return k(csrc_a, cdl_a, hlen_a, hstart_a, Bm, Am)


def _full(shape):
    return pl.BlockSpec(shape, lambda *args: tuple(0 for _ in shape))


def _front(x, W1f, b1f, Wl0, bl0, Wd0, c0, Ws0):
    BM = 512

    def body(xr, w1r, b1r, wl0r, bl0r, wdr, cdr, wsr, ar, br, zr):
        h = jnp.dot(xr[...], w1r[...], preferred_element_type=jnp.float32, precision=lax.Precision.HIGHEST)
        h = h + b1r[...]
        h = _gelu(h)
        z = jnp.dot(h, wl0r[...], preferred_element_type=jnp.float32, precision=lax.Precision.HIGHEST) + bl0r[...]
        zr[...] = _gelu(z)
        ar[...] = jnp.dot(h, wdr[...], preferred_element_type=jnp.float32, precision=lax.Precision.HIGHEST) + cdr[...]
        br[...] = jnp.dot(h, wsr[...], preferred_element_type=jnp.float32, precision=lax.Precision.HIGHEST)

    row = lambda d: pl.BlockSpec((BM, d), lambda i: (i, 0))
    return pl.pallas_call(
        body,
        grid=(NPAD // BM,),
        in_specs=[row(2048), _full((2048, 64)), _full((1, 64)),
                  _full((64, 32)), _full((1, 32)), _full((64, 48)),
                  _full((1, 48)), _full((64, 48))],
        out_specs=[row(48), row(48), row(32)],
        out_shape=[jax.ShapeDtypeStruct((NPAD, 48), jnp.float32),
                   jax.ShapeDtypeStruct((NPAD, 48), jnp.float32),
                   jax.ShapeDtypeStruct((NPAD, 32), jnp.float32)],
    )(x, W1f, b1f, Wl0, bl0, Wd0, c0, Ws0)


def _mid(h, zs, Wd, c, Ws, Wlin, blin, din, dout):
    BM = 512

    def body(hr, zr_in, wdr, cdr, wsr, wlr, blr, ar, br, zr):
        hh = hr[...]
        ar[...] = jnp.dot(hh, wdr[...], preferred_element_type=jnp.float32, precision=lax.Precision.HIGHEST) + cdr[...]
        br[...] = jnp.dot(hh, wsr[...], preferred_element_type=jnp.float32, precision=lax.Precision.HIGHEST)
        zr[...] = zr_in[...] + jnp.dot(hh, wlr[...], preferred_element_type=jnp.float32, precision=lax.Precision.HIGHEST) + blr[...]

    row = lambda d: pl.BlockSpec((BM, d), lambda i: (i, 0))
    return pl.pallas_call(
        body,
        grid=(NPAD // BM,),
        in_specs=[row(din), row(32), _full((din, dout)), _full((1, dout)),
                  _full((din, dout)), _full((din, 32)), _full((1, 32))],
        out_specs=[row(dout), row(dout), row(32)],
        out_shape=[jax.ShapeDtypeStruct((NPAD, dout), jnp.float32),
                   jax.ShapeDtypeStruct((NPAD, dout), jnp.float32),
                   jax.ShapeDtypeStruct((NPAD, 32), jnp.float32)],
    )(h, zs, Wd, c, Ws, Wlin, blin)


def _final(zs, h3, batch2d, x_omic, fp):
    def body(zsr, h3r, batr, xor, lwr, lbr,
             m0w, m0b, m1w, m1b, m2w, m2b, m3w, m3b,
             h1w, h1b, z1w, z1b, ow1, ob1,
             h2w, h2b, z2w, z2b, ow2, ob2,
             e1w, e1b, e2w, e2b, cw, cb, feat, haz):
        zs3 = zsr[...] + jnp.dot(h3r[...], lwr[...],
                                 preferred_element_type=jnp.float32, precision=lax.Precision.HIGHEST) + lbr[...]
        rowid = lax.broadcasted_iota(jnp.int32, (NPAD, 32), 0)
        zs3 = jnp.where(rowid < N_NODES, zs3, 0.0)
        gi = lax.broadcasted_iota(jnp.int32, (NG, NPAD), 0)
        oh = (batr[...] == gi).astype(jnp.float32)
        sums = jnp.dot(oh, zs3, preferred_element_type=jnp.float32, precision=lax.Precision.HIGHEST)
        cnt = jnp.sum(oh, axis=1, keepdims=True)
        v1 = sums / jnp.maximum(cnt, 1.0)

        ho = xor[...]
        for wr, br_ in ((m0w, m0b), (m1w, m1b), (m2w, m2b), (m3w, m3b)):
            t = jnp.dot(ho, wr[...], preferred_element_type=jnp.float32, precision=lax.Precision.HIGHEST) + br_[...]
            ho = jnp.where(t > 0, t, jnp.exp(jnp.minimum(t, 0.0)) - 1.0)
        v2 = ho

        def bilinear(va, w3r, br_, vb):
            W3 = w3r[...]
            acc = jnp.zeros((NG, 32), jnp.float32)
            for o in range(32):
                t = jnp.dot(va, W3[o], preferred_element_type=jnp.float32, precision=lax.Precision.HIGHEST)
                s = jnp.sum(t * vb, axis=1, keepdims=True)
                onec = (lax.broadcasted_iota(jnp.int32, (1, 32), 1) == o
                        ).astype(jnp.float32)
                acc = acc + s * onec
            return acc + br_[...]

        hh1 = jnp.maximum(jnp.dot(v1, h1w[...],
                                  preferred_element_type=jnp.float32, precision=lax.Precision.HIGHEST) + h1b[...], 0.0)
        z1 = bilinear(v1, z1w, z1b, v2)
        sig1 = 1.0 / (1.0 + jnp.exp(-z1))
        oo1 = jnp.maximum(jnp.dot(sig1 * hh1, ow1[...],
                                  preferred_element_type=jnp.float32, precision=lax.Precision.HIGHEST) + ob1[...], 0.0)
        hh2 = jnp.maximum(jnp.dot(v2, h2w[...],
                                  preferred_element_type=jnp.float32, precision=lax.Precision.HIGHEST) + h2b[...], 0.0)
        z2 = bilinear(v1, z2w, z2b, v2)
        sig2 = 1.0 / (1.0 + jnp.exp(-z2))
        oo2 = jnp.maximum(jnp.dot(sig2 * hh2, ow2[...],
                                  preferred_element_type=jnp.float32, precision=lax.Precision.HIGHEST) + ob2[...], 0.0)

        ones = jnp.ones((NG, 1), jnp.float32)
        E1 = e1w[...]
        acc = jnp.zeros((NG, 64), jnp.float32)
        for i in range(33):
            col = oo1[:, i:i + 1] if i < 32 else ones
            Ei = E1[33 * i:33 * i + 33, :]
            part = jnp.dot(oo2, Ei[:32, :],
                           preferred_element_type=jnp.float32, precision=lax.Precision.HIGHEST) + Ei[32:33, :]
            acc = acc + col * part
        out1 = jnp.maximum(acc + e1b[...], 0.0)

        E2 = e2w[...]
        pre = (jnp.dot(out1, E2[0:64], preferred_element_type=jnp.float32, precision=lax.Precision.HIGHEST)
               + jnp.dot(oo1, E2[64:96], preferred_element_type=jnp.float32, precision=lax.Precision.HIGHEST)
               + E2[96:97]
               + jnp.dot(oo2, E2[97:129], preferred_element_type=jnp.float32, precision=lax.Precision.HIGHEST)
               + E2[129:130] + e2b[...])
        feats = jnp.maximum(pre, 0.0)
        feat[...] = feats
        haz[...] = jnp.dot(feats, cw[...], preferred_element_type=jnp.float32, precision=lax.Precision.HIGHEST) + cb[...]

    p = fp
    ins = [zs, h3, batch2d, x_omic, p["lw"], p["lb"],
           p["m0w"], p["m0b"], p["m1w"], p["m1b"], p["m2w"], p["m2b"],
           p["m3w"], p["m3b"],
           p["h1w"], p["h1b"], p["z1w"], p["z1b"], p["ow1"], p["ob1"],
           p["h2w"], p["h2b"], p["z2w"], p["z2b"], p["ow2"], p["ob2"],
           p["e1w"], p["e1b"], p["e2w"], p["e2b"], p["cw"], p["cb"]]
    return pl.pallas_call(
        body,
        in_specs=[_full(v.shape) for v in ins],
        out_specs=[_full((NG, 64)), _full((NG, 1))],
        out_shape=[jax.ShapeDtypeStruct((NG, 64), jnp.float32),
                   jax.ShapeDtypeStruct((NG, 1), jnp.float32)],
    )(*ins)


def kernel(x, edge_index, batch, x_omic, params):
    p = params
    s0 = p["bn0_g"] * _BN
    W1f = p["first_h"]["W"] * s0[None, :]
    b1f = (p["first_h"]["b"] * s0 + p["bn0_b"])[None, :]
    Wl0 = p["lin0"]["W"]
    bl0 = p["lin0"]["b"][None, :]

    Wd, Cc, Ws = [], [], []
    dims = [(64, 48), (48, 32), (32, 32)]
    for l, (din, dout) in enumerate(dims):
        cp = p["convs"][l]
        W = cp["lin"]["W"]
        Wt, Wb = W[:din], W[din:]
        s = cp["bn_g"] * _BN
        Wd.append((Wt - Wb) * s[None, :])
        Ws.append(Wb * s[None, :])
        Cc.append((cp["lin"]["b"] * s + cp["bn_b"])[None, :])

    A0, B0, Zs = _front(x, W1f, b1f, Wl0, bl0, Wd[0], Cc[0], Ws[0])
    csrc, cdl, hlen, hstart = _bin_edges(edge_index)

    h1 = _seg_max(csrc, cdl, hlen, hstart, B0, A0, 48)
    A1, B1, Zs = _mid(h1, Zs, Wd[1], Cc[1], Ws[1],
                      p["lins"][0]["W"], p["lins"][0]["b"][None, :], 48, 32)
    h2 = _seg_max(csrc, cdl, hlen, hstart, B1, A1, 32)
    A2, B2, Zs = _mid(h2, Zs, Wd[2], Cc[2], Ws[2],
                      p["lins"][1]["W"], p["lins"][1]["b"][None, :], 32, 32)
    h3 = _seg_max(csrc, cdl, hlen, hstart, B2, A2, 32)

    batch2d = jnp.concatenate(
        [batch, jnp.full((NPAD - N_NODES,), NG, jnp.int32)]).reshape(1, NPAD)
    fp = {
        "lw": p["lins"][2]["W"], "lb": p["lins"][2]["b"][None, :],
        "m0w": p["omic"][0]["W"], "m0b": p["omic"][0]["b"][None, :],
        "m1w": p["omic"][1]["W"], "m1b": p["omic"][1]["b"][None, :],
        "m2w": p["omic"][2]["W"], "m2b": p["omic"][2]["b"][None, :],
        "m3w": p["omic"][3]["W"], "m3b": p["omic"][3]["b"][None, :],
        "h1w": p["h1"]["W"], "h1b": p["h1"]["b"][None, :],
        "z1w": p["z1_W"], "z1b": p["z1_b"][None, :],
        "ow1": p["o1"]["W"], "ob1": p["o1"]["b"][None, :],
        "h2w": p["h2"]["W"], "h2b": p["h2"]["b"][None, :],
        "z2w": p["z2_W"], "z2b": p["z2_b"][None, :],
        "ow2": p["o2"]["W"], "ob2": p["o2"]["b"][None, :],
        "e1w": p["enc1"]["W"], "e1b": p["enc1"]["b"][None, :],
        "e2w": p["enc2"]["W"], "e2b": p["enc2"]["b"][None, :],
        "cw": p["clf"]["W"], "cb": p["clf"]["b"][None, :],
    }
    features, hazard = _final(Zs, h3, batch2d, x_omic, fp)
    return features, hazard


# speedup vs baseline: 5.0023x; 1.0541x over previous
import functools

import jax
import jax.numpy as jnp
import numpy as np
from jax import lax
from jax.experimental import pallas as pl
from jax.experimental.pallas import tpu as pltpu
from jax.experimental.pallas import tpu_sc as plsc

_BN = 1.0 / np.sqrt(1.0 + 1e-5)

N_NODES = 10000
NT = 32
NB = 320
NPAD = NT * NB
NE = 640000
CHUNK = NE // NT
LROWS = 188
CELL_ROWS = NT * LROWS + 8
BLK = 1024
NG = 8

_SC_PARAMS = pltpu.CompilerParams(
    use_tc_tiling_on_sc=False, needs_layout_passes=False)


def _gelu(x):
    return 0.5 * x * (1.0 + lax.erf(x * np.float32(0.7071067811865476)))


def _mesh():
    return plsc.VectorSubcoreMesh(core_axis_name="c", subcore_axis_name="s")


def _wid():
    return lax.axis_index("s") * 2 + lax.axis_index("c")


def _bin_edges(edge_index):
    @functools.partial(
        pl.kernel,
        mesh=_mesh(),
        compiler_params=_SC_PARAMS,
        out_type=(
            jax.ShapeDtypeStruct((CELL_ROWS, 128), jnp.int32),
            jax.ShapeDtypeStruct((CELL_ROWS, 128), jnp.int32),
            jax.ShapeDtypeStruct((NT, NT), jnp.int32),
            jax.ShapeDtypeStruct((NT, NT), jnp.int32),
        ),
        scratch_types=[
            pltpu.VMEM((2000,), jnp.int32),
            pltpu.VMEM((2000,), jnp.int32),
            pltpu.VMEM((LROWS, 128), jnp.int32),
            pltpu.VMEM((LROWS, 128), jnp.int32),
            pltpu.VMEM((32,), jnp.int32),
            pltpu.VMEM((32,), jnp.int32),
            pltpu.VMEM((32,), jnp.int32),
            pltpu.VMEM((32,), jnp.int32),
            pltpu.VMEM((8, 128), jnp.int32),
            pltpu.VMEM((8, 128), jnp.int32),
        ],
    )
    def k(ei, csrc, cdl, hlen, hstart, sbuf, dbuf, lsrc, ldloc, histv,
          lbase, len16v, startv, pad_src, pad_dl):
        w = _wid()
        ebase = w * CHUNK
        z16 = jnp.zeros((16,), jnp.int32)
        s16full = jnp.full((16,), NB, jnp.int32)
        histv[pl.ds(0, 16)] = z16
        histv[pl.ds(16, 16)] = z16

        for blk in range(CHUNK // 2000):
            pltpu.sync_copy(ei.at[1, pl.ds(ebase + 2000 * blk, 2000)], dbuf)

            def h_body(g, _):
                d16 = dbuf[pl.ds(16 * g, 16)]
                b16 = (d16 * 26215) >> 23
                cnt, last = plsc.scan_count(b16)
                plsc.addupdate_scatter(histv, [b16], cnt, mask=last)
                return 0

            lax.fori_loop(0, 125, h_body, 0)

        h0 = histv[pl.ds(0, 16)]
        h1 = histv[pl.ds(16, 16)]
        r0 = (h0 + 127) & ~127
        r1 = (h1 + 127) & ~127
        c0 = plsc.cumsum(r0)
        c1 = plsc.cumsum(r1)
        tot0 = jnp.max(c0)
        s0 = c0 - r0
        s1 = c1 - r1 + tot0
        lbase[pl.ds(0, 16)] = s0
        lbase[pl.ds(16, 16)] = s1
        startv[pl.ds(0, 16)] = s0 >> 7
        startv[pl.ds(16, 16)] = s1 >> 7
        len16v[pl.ds(0, 16)] = (h0 + 15) & ~15
        len16v[pl.ds(16, 16)] = (h1 + 15) & ~15

        def sent(r, _):
            for j in range(8):
                lsrc[r, pl.ds(16 * j, 16)] = z16
                ldloc[r, pl.ds(16 * j, 16)] = s16full
            return 0

        lax.fori_loop(0, LROWS, sent, 0)

        for blk in range(CHUNK // 2000):
            pltpu.sync_copy(ei.at[0, pl.ds(ebase + 2000 * blk, 2000)], sbuf)
            pltpu.sync_copy(ei.at[1, pl.ds(ebase + 2000 * blk, 2000)], dbuf)

            def p_body(g, _):
                sv = sbuf[pl.ds(16 * g, 16)]
                d16 = dbuf[pl.ds(16 * g, 16)]
                b16 = (d16 * 26215) >> 23
                dl16 = d16 - b16 * NB
                cnt, last = plsc.scan_count(b16)
                base16 = plsc.load_gather(lbase, [b16])
                pos = base16 + cnt - 1
                plsc.store_scatter(lsrc, [pos >> 7, pos & 127], sv)
                plsc.store_scatter(ldloc, [pos >> 7, pos & 127], dl16)
                plsc.addupdate_scatter(lbase, [b16], cnt, mask=last)
                return 0

            lax.fori_loop(0, 125, p_body, 0)

        pltpu.sync_copy(lsrc, csrc.at[pl.ds(LROWS * w, LROWS)])
        pltpu.sync_copy(ldloc, cdl.at[pl.ds(LROWS * w, LROWS)])
        pltpu.sync_copy(len16v, hlen.at[w])
        pltpu.sync_copy(startv, hstart.at[w])

        def padr(r, _):
            for j in range(8):
                pad_src[r, pl.ds(16 * j, 16)] = z16
                pad_dl[r, pl.ds(16 * j, 16)] = s16full
            return 0

        lax.fori_loop(0, 8, padr, 0)

        @pl.when(w == 0)
        def _():
            pltpu.sync_copy(pad_src, csrc.at[pl.ds(NT * LROWS, 8)])
            pltpu.sync_copy(pad_dl, cdl.at[pl.ds(NT * LROWS, 8)])

    return k(edge_index)


def _seg_max(csrc_a, cdl_a, hlen_a, hstart_a, Bm, Am, D):
    nf = D // 16
    K = 4
    blk = 512 if D == 48 else 1024
    nsub = blk // 128

    @functools.partial(
        pl.kernel,
        mesh=_mesh(),
        compiler_params=_SC_PARAMS,
        out_type=jax.ShapeDtypeStruct((NPAD, D), jnp.float32),
        scratch_types=[
            pltpu.VMEM((nsub, 128), jnp.int32),
            pltpu.VMEM((nsub, 128), jnp.int32),
            pltpu.VMEM((blk, D), jnp.float32),
            [pltpu.VMEM((NB + 8, D), jnp.float32) for _ in range(K)],
            pltpu.VMEM((NB, D), jnp.float32),
            pltpu.VMEM((NT, NT), jnp.int32),
            pltpu.VMEM((NT, NT), jnp.int32),
            pltpu.SemaphoreType.DMA,
            pltpu.SemaphoreType.DMA,
        ],
    )
    def k(csrc, cdl, hlen, hstart, B_, A_, hout, idx2d, dloc2d, rows, tbls,
          abuf, hbuf, sbufr, sem, sem2):
        w = _wid()
        acp = pltpu.async_copy(A_.at[pl.ds(NB * w, NB)], abuf, sem2)
        pltpu.sync_copy(hlen, hbuf)
        pltpu.sync_copy(hstart, sbufr)

        neg = jnp.full((16,), -jnp.inf, jnp.float32)

        def initr(r, _):
            for tb in tbls:
                for f in range(nf):
                    tb[r, pl.ds(16 * f, 16)] = neg
            return 0

        lax.fori_loop(0, NB + 8, initr, 0)

        woff = (w >> 4) << 4
        wlane = w & 15
        lanes = lax.iota(jnp.int32, 16)

        def per_t(t, _):
            hv = hbuf[t, pl.ds(woff, 16)]
            len16 = jnp.max(jnp.where(lanes == wlane, hv, 0))
            sv = sbufr[t, pl.ds(woff, 16)]
            srow = jnp.max(jnp.where(lanes == wlane, sv, 0))
            baser = LROWS * t + srow
            nblk = (len16 + (blk - 1)) // blk

            def per_blk(kk, _):
                pltpu.sync_copy(csrc.at[pl.ds(baser + nsub * kk, nsub)], idx2d)
                pltpu.sync_copy(cdl.at[pl.ds(baser + nsub * kk, nsub)], dloc2d)
                cps = [
                    pltpu.async_copy(B_.at[idx2d.at[j]],
                                     rows.at[pl.ds(128 * j, 128)], sem)
                    for j in range(nsub)
                ]
                for cp in cps:
                    cp.wait()
                rem = len16 - kk * blk
                ngr = jnp.minimum(rem, blk) >> 4

                def per_g(g, _):
                    dv = dloc2d[g >> 3, pl.ds((g & 7) * 16, 16)]
                    for j in range(16):
                        d = dv[j]
                        e = g * 16 + j
                        tb = tbls[j % K]
                        for f in range(nf):
                            sl = pl.ds(16 * f, 16)
                            tb[d, sl] = jnp.maximum(tb[d, sl], rows[e, sl])
                    return 0

                lax.fori_loop(0, ngr, per_g, 0)
                return 0

            lax.fori_loop(0, nblk, per_blk, 0)
            return 0

        lax.fori_loop(0, NT, per_t, 0)
        acp.wait()

        zeros = jnp.zeros((16,), jnp.float32)

        def finr(r, _):
            for f in range(nf):
                sl = pl.ds(16 * f, 16)
                m = tbls[0][r, sl]
                for tb in tbls[1:]:
                    m = jnp.maximum(m, tb[r, sl])
                abuf[r, sl] = jnp.where(m > -3.0e38, abuf[r, sl] + m, zeros)
            return 0

        lax.fori_loop(0, NB, finr, 0)
        pltpu.sync_copy(abuf, hout.at[pl.ds(NB * w, NB)])

    return k(csrc_a, cdl_a, hlen_a, hstart_a, Bm, Am)


def _full(shape):
    return pl.BlockSpec(shape, lambda *args: tuple(0 for _ in shape))


def _front(x, W1f, b1f, Wl0, bl0, Wd0, c0, Ws0):
    BM = 512

    def body(xr, w1r, b1r, wl0r, bl0r, wdr, cdr, wsr, ar, br, zr):
        h = jnp.dot(xr[...], w1r[...], preferred_element_type=jnp.float32, precision=lax.Precision.HIGHEST)
        h = h + b1r[...]
        h = _gelu(h)
        z = jnp.dot(h, wl0r[...], preferred_element_type=jnp.float32, precision=lax.Precision.HIGHEST) + bl0r[...]
        zr[...] = _gelu(z)
        ar[...] = jnp.dot(h, wdr[...], preferred_element_type=jnp.float32, precision=lax.Precision.HIGHEST) + cdr[...]
        br[...] = jnp.dot(h, wsr[...], preferred_element_type=jnp.float32, precision=lax.Precision.HIGHEST)

    row = lambda d: pl.BlockSpec((BM, d), lambda i: (i, 0))
    return pl.pallas_call(
        body,
        grid=(NPAD // BM,),
        in_specs=[row(2048), _full((2048, 64)), _full((1, 64)),
                  _full((64, 32)), _full((1, 32)), _full((64, 48)),
                  _full((1, 48)), _full((64, 48))],
        out_specs=[row(48), row(48), row(32)],
        out_shape=[jax.ShapeDtypeStruct((NPAD, 48), jnp.float32),
                   jax.ShapeDtypeStruct((NPAD, 48), jnp.float32),
                   jax.ShapeDtypeStruct((NPAD, 32), jnp.float32)],
    )(x, W1f, b1f, Wl0, bl0, Wd0, c0, Ws0)


def _mid(h, zs, Wd, c, Ws, Wlin, blin, din, dout):
    BM = 512

    def body(hr, zr_in, wdr, cdr, wsr, wlr, blr, ar, br, zr):
        hh = hr[...]
        ar[...] = jnp.dot(hh, wdr[...], preferred_element_type=jnp.float32, precision=lax.Precision.HIGHEST) + cdr[...]
        br[...] = jnp.dot(hh, wsr[...], preferred_element_type=jnp.float32, precision=lax.Precision.HIGHEST)
        zr[...] = zr_in[...] + jnp.dot(hh, wlr[...], preferred_element_type=jnp.float32, precision=lax.Precision.HIGHEST) + blr[...]

    row = lambda d: pl.BlockSpec((BM, d), lambda i: (i, 0))
    return pl.pallas_call(
        body,
        grid=(NPAD // BM,),
        in_specs=[row(din), row(32), _full((din, dout)), _full((1, dout)),
                  _full((din, dout)), _full((din, 32)), _full((1, 32))],
        out_specs=[row(dout), row(dout), row(32)],
        out_shape=[jax.ShapeDtypeStruct((NPAD, dout), jnp.float32),
                   jax.ShapeDtypeStruct((NPAD, dout), jnp.float32),
                   jax.ShapeDtypeStruct((NPAD, 32), jnp.float32)],
    )(h, zs, Wd, c, Ws, Wlin, blin)


def _final(zs, h3, batch2d, x_omic, fp):
    def body(zsr, h3r, batr, xor, lwr, lbr,
             m0w, m0b, m1w, m1b, m2w, m2b, m3w, m3b,
             h1w, h1b, z1w, z1b, ow1, ob1,
             h2w, h2b, z2w, z2b, ow2, ob2,
             e1w, e1b, e2w, e2b, cw, cb, feat, haz):
        zs3 = zsr[...] + jnp.dot(h3r[...], lwr[...],
                                 preferred_element_type=jnp.float32, precision=lax.Precision.HIGHEST) + lbr[...]
        rowid = lax.broadcasted_iota(jnp.int32, (NPAD, 32), 0)
        zs3 = jnp.where(rowid < N_NODES, zs3, 0.0)
        gi = lax.broadcasted_iota(jnp.int32, (NG, NPAD), 0)
        oh = (batr[...] == gi).astype(jnp.float32)
        sums = jnp.dot(oh, zs3, preferred_element_type=jnp.float32, precision=lax.Precision.HIGHEST)
        cnt = jnp.sum(oh, axis=1, keepdims=True)
        v1 = sums / jnp.maximum(cnt, 1.0)

        ho = xor[...]
        for wr, br_ in ((m0w, m0b), (m1w, m1b), (m2w, m2b), (m3w, m3b)):
            t = jnp.dot(ho, wr[...], preferred_element_type=jnp.float32, precision=lax.Precision.HIGHEST) + br_[...]
            ho = jnp.where(t > 0, t, jnp.exp(jnp.minimum(t, 0.0)) - 1.0)
        v2 = ho

        def bilinear(va, w3r, br_, vb):
            W3 = w3r[...]
            acc = jnp.zeros((NG, 32), jnp.float32)
            for o in range(32):
                t = jnp.dot(va, W3[o], preferred_element_type=jnp.float32, precision=lax.Precision.HIGHEST)
                s = jnp.sum(t * vb, axis=1, keepdims=True)
                onec = (lax.broadcasted_iota(jnp.int32, (1, 32), 1) == o
                        ).astype(jnp.float32)
                acc = acc + s * onec
            return acc + br_[...]

        hh1 = jnp.maximum(jnp.dot(v1, h1w[...],
                                  preferred_element_type=jnp.float32, precision=lax.Precision.HIGHEST) + h1b[...], 0.0)
        z1 = bilinear(v1, z1w, z1b, v2)
        sig1 = 1.0 / (1.0 + jnp.exp(-z1))
        oo1 = jnp.maximum(jnp.dot(sig1 * hh1, ow1[...],
                                  preferred_element_type=jnp.float32, precision=lax.Precision.HIGHEST) + ob1[...], 0.0)
        hh2 = jnp.maximum(jnp.dot(v2, h2w[...],
                                  preferred_element_type=jnp.float32, precision=lax.Precision.HIGHEST) + h2b[...], 0.0)
        z2 = bilinear(v1, z2w, z2b, v2)
        sig2 = 1.0 / (1.0 + jnp.exp(-z2))
        oo2 = jnp.maximum(jnp.dot(sig2 * hh2, ow2[...],
                                  preferred_element_type=jnp.float32, precision=lax.Precision.HIGHEST) + ob2[...], 0.0)

        ones = jnp.ones((NG, 1), jnp.float32)
        E1 = e1w[...]
        acc = jnp.zeros((NG, 64), jnp.float32)
        for i in range(33):
            col = oo1[:, i:i + 1] if i < 32 else ones
            Ei = E1[33 * i:33 * i + 33, :]
            part = jnp.dot(oo2, Ei[:32, :],
                           preferred_element_type=jnp.float32, precision=lax.Precision.HIGHEST) + Ei[32:33, :]
            acc = acc + col * part
        out1 = jnp.maximum(acc + e1b[...], 0.0)

        E2 = e2w[...]
        pre = (jnp.dot(out1, E2[0:64], preferred_element_type=jnp.float32, precision=lax.Precision.HIGHEST)
               + jnp.dot(oo1, E2[64:96], preferred_element_type=jnp.float32, precision=lax.Precision.HIGHEST)
               + E2[96:97]
               + jnp.dot(oo2, E2[97:129], preferred_element_type=jnp.float32, precision=lax.Precision.HIGHEST)
               + E2[129:130] + e2b[...])
        feats = jnp.maximum(pre, 0.0)
        feat[...] = feats
        haz[...] = jnp.dot(feats, cw[...], preferred_element_type=jnp.float32, precision=lax.Precision.HIGHEST) + cb[...]

    p = fp
    ins = [zs, h3, batch2d, x_omic, p["lw"], p["lb"],
           p["m0w"], p["m0b"], p["m1w"], p["m1b"], p["m2w"], p["m2b"],
           p["m3w"], p["m3b"],
           p["h1w"], p["h1b"], p["z1w"], p["z1b"], p["ow1"], p["ob1"],
           p["h2w"], p["h2b"], p["z2w"], p["z2b"], p["ow2"], p["ob2"],
           p["e1w"], p["e1b"], p["e2w"], p["e2b"], p["cw"], p["cb"]]
    return pl.pallas_call(
        body,
        in_specs=[_full(v.shape) for v in ins],
        out_specs=[_full((NG, 64)), _full((NG, 1))],
        out_shape=[jax.ShapeDtypeStruct((NG, 64), jnp.float32),
                   jax.ShapeDtypeStruct((NG, 1), jnp.float32)],
    )(*ins)


def kernel(x, edge_index, batch, x_omic, params):
    p = params
    s0 = p["bn0_g"] * _BN
    W1f = p["first_h"]["W"] * s0[None, :]
    b1f = (p["first_h"]["b"] * s0 + p["bn0_b"])[None, :]
    Wl0 = p["lin0"]["W"]
    bl0 = p["lin0"]["b"][None, :]

    Wd, Cc, Ws = [], [], []
    dims = [(64, 48), (48, 32), (32, 32)]
    for l, (din, dout) in enumerate(dims):
        cp = p["convs"][l]
        W = cp["lin"]["W"]
        Wt, Wb = W[:din], W[din:]
        s = cp["bn_g"] * _BN
        Wd.append((Wt - Wb) * s[None, :])
        Ws.append(Wb * s[None, :])
        Cc.append((cp["lin"]["b"] * s + cp["bn_b"])[None, :])

    A0, B0, Zs = _front(x, W1f, b1f, Wl0, bl0, Wd[0], Cc[0], Ws[0])
    csrc, cdl, hlen, hstart = _bin_edges(edge_index)

    h1 = _seg_max(csrc, cdl, hlen, hstart, B0, A0, 48)
    A1, B1, Zs = _mid(h1, Zs, Wd[1], Cc[1], Ws[1],
                      p["lins"][0]["W"], p["lins"][0]["b"][None, :], 48, 32)
    h2 = _seg_max(csrc, cdl, hlen, hstart, B1, A1, 32)
    A2, B2, Zs = _mid(h2, Zs, Wd[2], Cc[2], Ws[2],
                      p["lins"][1]["W"], p["lins"][1]["b"][None, :], 32, 32)
    h3 = _seg_max(csrc, cdl, hlen, hstart, B2, A2, 32)

    batch2d = jnp.concatenate(
        [batch, jnp.full((NPAD - N_NODES,), NG, jnp.int32)]).reshape(1, NPAD)
    fp = {
        "lw": p["lins"][2]["W"], "lb": p["lins"][2]["b"][None, :],
        "m0w": p["omic"][0]["W"], "m0b": p["omic"][0]["b"][None, :],
        "m1w": p["omic"][1]["W"], "m1b": p["omic"][1]["b"][None, :],
        "m2w": p["omic"][2]["W"], "m2b": p["omic"][2]["b"][None, :],
        "m3w": p["omic"][3]["W"], "m3b": p["omic"][3]["b"][None, :],
        "h1w": p["h1"]["W"], "h1b": p["h1"]["b"][None, :],
        "z1w": p["z1_W"], "z1b": p["z1_b"][None, :],
        "ow1": p["o1"]["W"], "ob1": p["o1"]["b"][None, :],
        "h2w": p["h2"]["W"], "h2b": p["h2"]["b"][None, :],
        "z2w": p["z2_W"], "z2b": p["z2_b"][None, :],
        "ow2": p["o2"]["W"], "ob2": p["o2"]["b"][None, :],
        "e1w": p["enc1"]["W"], "e1b": p["enc1"]["b"][None, :],
        "e2w": p["enc2"]["W"], "e2b": p["enc2"]["b"][None, :],
        "cw": p["clf"]["W"], "cb": p["clf"]["b"][None, :],
    }
    features, hazard = _final(Zs, h3, batch2d, x_omic, fp)
    return features, hazard


# speedup vs baseline: 5.2818x; 1.0559x over previous
import functools

import jax
import jax.numpy as jnp
import numpy as np
from jax import lax
from jax.experimental import pallas as pl
from jax.experimental.pallas import tpu as pltpu
from jax.experimental.pallas import tpu_sc as plsc

_BN = 1.0 / np.sqrt(1.0 + 1e-5)

N_NODES = 10000
NT = 32
NB = 320
NPAD = NT * NB
NE = 640000
CHUNK = NE // NT
LROWS = 188
CELL_ROWS = NT * LROWS + 8
BLK = 1024
NG = 8

_SC_PARAMS = pltpu.CompilerParams(
    use_tc_tiling_on_sc=False, needs_layout_passes=False)


def _gelu(x):
    return 0.5 * x * (1.0 + lax.erf(x * np.float32(0.7071067811865476)))


def _mesh():
    return plsc.VectorSubcoreMesh(core_axis_name="c", subcore_axis_name="s")


def _wid():
    return lax.axis_index("s") * 2 + lax.axis_index("c")


def _bin_edges(edge_index):
    @functools.partial(
        pl.kernel,
        mesh=_mesh(),
        compiler_params=_SC_PARAMS,
        out_type=(
            jax.ShapeDtypeStruct((CELL_ROWS, 128), jnp.int32),
            jax.ShapeDtypeStruct((CELL_ROWS, 128), jnp.int32),
            jax.ShapeDtypeStruct((NT, NT), jnp.int32),
            jax.ShapeDtypeStruct((NT, NT), jnp.int32),
        ),
        scratch_types=[
            pltpu.VMEM((2000,), jnp.int32),
            pltpu.VMEM((2000,), jnp.int32),
            pltpu.VMEM((LROWS, 128), jnp.int32),
            pltpu.VMEM((LROWS, 128), jnp.int32),
            pltpu.VMEM((32,), jnp.int32),
            pltpu.VMEM((32,), jnp.int32),
            pltpu.VMEM((32,), jnp.int32),
            pltpu.VMEM((32,), jnp.int32),
            pltpu.VMEM((8, 128), jnp.int32),
            pltpu.VMEM((8, 128), jnp.int32),
        ],
    )
    def k(ei, csrc, cdl, hlen, hstart, sbuf, dbuf, lsrc, ldloc, histv,
          lbase, len16v, startv, pad_src, pad_dl):
        w = _wid()
        ebase = w * CHUNK
        z16 = jnp.zeros((16,), jnp.int32)
        s16full = jnp.full((16,), NB, jnp.int32)
        histv[pl.ds(0, 16)] = z16
        histv[pl.ds(16, 16)] = z16

        for blk in range(CHUNK // 2000):
            pltpu.sync_copy(ei.at[1, pl.ds(ebase + 2000 * blk, 2000)], dbuf)

            def h_body(g, _):
                d16 = dbuf[pl.ds(16 * g, 16)]
                b16 = (d16 * 26215) >> 23
                cnt, last = plsc.scan_count(b16)
                plsc.addupdate_scatter(histv, [b16], cnt, mask=last)
                return 0

            lax.fori_loop(0, 125, h_body, 0)

        h0 = histv[pl.ds(0, 16)]
        h1 = histv[pl.ds(16, 16)]
        r0 = (h0 + 127) & ~127
        r1 = (h1 + 127) & ~127
        c0 = plsc.cumsum(r0)
        c1 = plsc.cumsum(r1)
        tot0 = jnp.max(c0)
        s0 = c0 - r0
        s1 = c1 - r1 + tot0
        lbase[pl.ds(0, 16)] = s0
        lbase[pl.ds(16, 16)] = s1
        startv[pl.ds(0, 16)] = s0 >> 7
        startv[pl.ds(16, 16)] = s1 >> 7
        len16v[pl.ds(0, 16)] = (h0 + 15) & ~15
        len16v[pl.ds(16, 16)] = (h1 + 15) & ~15

        def sent(r, _):
            for j in range(8):
                lsrc[r, pl.ds(16 * j, 16)] = z16
                ldloc[r, pl.ds(16 * j, 16)] = s16full
            return 0

        lax.fori_loop(0, LROWS, sent, 0)

        for blk in range(CHUNK // 2000):
            pltpu.sync_copy(ei.at[0, pl.ds(ebase + 2000 * blk, 2000)], sbuf)
            pltpu.sync_copy(ei.at[1, pl.ds(ebase + 2000 * blk, 2000)], dbuf)

            def p_body(g, _):
                sv = sbuf[pl.ds(16 * g, 16)]
                d16 = dbuf[pl.ds(16 * g, 16)]
                b16 = (d16 * 26215) >> 23
                dl16 = d16 - b16 * NB
                cnt, last = plsc.scan_count(b16)
                base16 = plsc.load_gather(lbase, [b16])
                pos = base16 + cnt - 1
                plsc.store_scatter(lsrc, [pos >> 7, pos & 127], sv)
                plsc.store_scatter(ldloc, [pos >> 7, pos & 127], dl16)
                plsc.addupdate_scatter(lbase, [b16], cnt, mask=last)
                return 0

            lax.fori_loop(0, 125, p_body, 0)

        pltpu.sync_copy(lsrc, csrc.at[pl.ds(LROWS * w, LROWS)])
        pltpu.sync_copy(ldloc, cdl.at[pl.ds(LROWS * w, LROWS)])
        pltpu.sync_copy(len16v, hlen.at[w])
        pltpu.sync_copy(startv, hstart.at[w])

        def padr(r, _):
            for j in range(8):
                pad_src[r, pl.ds(16 * j, 16)] = z16
                pad_dl[r, pl.ds(16 * j, 16)] = s16full
            return 0

        lax.fori_loop(0, 8, padr, 0)

        @pl.when(w == 0)
        def _():
            pltpu.sync_copy(pad_src, csrc.at[pl.ds(NT * LROWS, 8)])
            pltpu.sync_copy(pad_dl, cdl.at[pl.ds(NT * LROWS, 8)])

    return k(edge_index)


def _seg_max(csrc_a, cdl_a, hlen_a, hstart_a, Bm, Am, D):
    nf = D // 16
    K = 2
    blk = 512 if D == 48 else 1024
    nsub = blk // 128
    NDESC = ((NE // blk + NT + 31) // 16) * 16

    @functools.partial(
        pl.kernel,
        mesh=_mesh(),
        compiler_params=_SC_PARAMS,
        out_type=jax.ShapeDtypeStruct((NPAD, D), jnp.float32),
        scratch_types=[
            [pltpu.VMEM((nsub, 128), jnp.int32) for _ in range(2)],
            [pltpu.VMEM((nsub, 128), jnp.int32) for _ in range(2)],
            [pltpu.VMEM((blk, D), jnp.float32) for _ in range(2)],
            [pltpu.VMEM((NB + 8, D), jnp.float32) for _ in range(K)],
            pltpu.VMEM((NB, D), jnp.float32),
            pltpu.VMEM((NT, NT), jnp.int32),
            pltpu.VMEM((NT, NT), jnp.int32),
            pltpu.VMEM((NDESC,), jnp.int32),
            pltpu.VMEM((NDESC,), jnp.int32),
            [pltpu.SemaphoreType.DMA for _ in range(2)],
            [pltpu.SemaphoreType.DMA for _ in range(2)],
            pltpu.SemaphoreType.DMA,
        ],
    )
    def k(csrc, cdl, hlen, hstart, B_, A_, hout, idx2d, dloc2d, rows, tbls,
          abuf, hbuf, sbufr, drow, dngr, isems, rsems, sem2):
        w = _wid()
        acp = pltpu.async_copy(A_.at[pl.ds(NB * w, NB)], abuf, sem2)
        pltpu.sync_copy(hlen, hbuf)
        pltpu.sync_copy(hstart, sbufr)

        neg = jnp.full((16,), -jnp.inf, jnp.float32)

        def initr(r, _):
            for tb in tbls:
                for f in range(nf):
                    tb[r, pl.ds(16 * f, 16)] = neg
            return 0

        lax.fori_loop(0, NB + 8, initr, 0)

        woff = (w >> 4) << 4
        wlane = w & 15
        lanes = lax.iota(jnp.int32, 16)

        def desc_t(t, cnt):
            hv = hbuf[t, pl.ds(woff, 16)]
            len16 = jnp.max(jnp.where(lanes == wlane, hv, 0))
            sv = sbufr[t, pl.ds(woff, 16)]
            srow = jnp.max(jnp.where(lanes == wlane, sv, 0))
            baser = LROWS * t + srow
            nblk = (len16 + (blk - 1)) // blk

            def desc_b(j0, cnt2):
                kvec = j0 * 16 + lanes
                rows16 = baser + kvec * nsub
                remv = len16 - kvec * blk
                ngr16 = jnp.minimum(remv, blk) >> 4
                mask = kvec < nblk
                plsc.store_scatter(drow, [cnt2 + lanes], rows16, mask=mask)
                plsc.store_scatter(dngr, [cnt2 + lanes], ngr16, mask=mask)
                return cnt2 + jnp.minimum(nblk - j0 * 16, 16)

            return lax.fori_loop(0, (nblk + 15) // 16, desc_b, cnt)

        nblocks = lax.fori_loop(0, NT, desc_t, jnp.int32(0))

        def desc_read(ref, i):
            v = ref[pl.ds((i >> 4) << 4, 16)]
            return jnp.max(jnp.where(lanes == (i & 15), v, 0))

        def fire_idx(i, s):
            r0 = desc_read(drow, i)
            pltpu.async_copy(csrc.at[pl.ds(r0, nsub)], idx2d[s], isems[s])
            pltpu.async_copy(cdl.at[pl.ds(r0, nsub)], dloc2d[s], isems[s])

        def wait_idx(i, s):
            r0 = desc_read(drow, i)
            pltpu.make_async_copy(csrc.at[pl.ds(r0, nsub)], idx2d[s], isems[s]).wait()
            pltpu.make_async_copy(cdl.at[pl.ds(r0, nsub)], dloc2d[s], isems[s]).wait()

        def fire_rows(s):
            for j in range(nsub):
                pltpu.async_copy(B_.at[idx2d[s].at[j]],
                                 rows[s].at[pl.ds(128 * j, 128)], rsems[s])

        def wait_rows(s):
            for j in range(nsub):
                pltpu.make_async_copy(B_.at[idx2d[s].at[j]],
                                      rows[s].at[pl.ds(128 * j, 128)],
                                      rsems[s]).wait()

        @pl.when(nblocks > 0)
        def _():
            fire_idx(jnp.int32(0), 0)
            wait_idx(jnp.int32(0), 0)
            fire_rows(0)

        @pl.when(nblocks > 1)
        def _():
            fire_idx(jnp.int32(1), 1)

        def do_block(i, s):
            wait_rows(s)

            @pl.when(i + 1 < nblocks)
            def _():
                wait_idx(i + 1, 1 - s)
                fire_rows(1 - s)

            ngr = desc_read(dngr, i)
            rws = rows[s]
            dl = dloc2d[s]

            def per_g(g, _):
                dv = dl[g >> 3, pl.ds((g & 7) * 16, 16)]
                for j in range(16):
                    d = dv[j]
                    e = g * 16 + j
                    tb = tbls[j % K]
                    for f in range(nf):
                        sl = pl.ds(16 * f, 16)
                        tb[d, sl] = jnp.maximum(tb[d, sl], rws[e, sl])
                return 0

            lax.fori_loop(0, ngr, per_g, 0)

            @pl.when(i + 2 < nblocks)
            def _():
                fire_idx(i + 2, s)

        def per_pair(p, _):
            for s in range(2):
                i = 2 * p + s

                @pl.when(i < nblocks)
                def _():
                    do_block(i, s)
            return 0

        lax.fori_loop(0, (nblocks + 1) >> 1, per_pair, 0)
        acp.wait()

        zeros = jnp.zeros((16,), jnp.float32)

        def finr(r, _):
            for f in range(nf):
                sl = pl.ds(16 * f, 16)
                m = tbls[0][r, sl]
                for tb in tbls[1:]:
                    m = jnp.maximum(m, tb[r, sl])
                abuf[r, sl] = jnp.where(m > -3.0e38, abuf[r, sl] + m, zeros)
            return 0

        lax.fori_loop(0, NB, finr, 0)
        pltpu.sync_copy(abuf, hout.at[pl.ds(NB * w, NB)])

    return k(csrc_a, cdl_a, hlen_a, hstart_a, Bm, Am)


def _full(shape):
    return pl.BlockSpec(shape, lambda *args: tuple(0 for _ in shape))


def _front(x, W1f, s0v, b1f, Wl0, bl0, Wd0, c0, Ws0):
    BM = 512

    def body(xr, w1r, s0r, b1r, wl0r, bl0r, wdr, cdr, wsr, ar, br, zr):
        h = jnp.dot(xr[...], w1r[...], preferred_element_type=jnp.float32)
        h = h * s0r[...] + b1r[...]
        h = _gelu(h)
        z = jnp.dot(h, wl0r[...], preferred_element_type=jnp.float32) + bl0r[...]
        zr[...] = _gelu(z)
        hq = h.astype(jnp.bfloat16).astype(jnp.float32)
        ar[...] = jnp.dot(hq, wdr[...], preferred_element_type=jnp.float32, precision=lax.Precision.HIGHEST) + cdr[...]
        br[...] = jnp.dot(hq, wsr[...], preferred_element_type=jnp.float32, precision=lax.Precision.HIGHEST)

    row = lambda d: pl.BlockSpec((BM, d), lambda i: (i, 0))
    return pl.pallas_call(
        body,
        grid=(NPAD // BM,),
        in_specs=[row(2048), _full((2048, 64)), _full((1, 64)), _full((1, 64)),
                  _full((64, 32)), _full((1, 32)), _full((64, 48)),
                  _full((1, 48)), _full((64, 48))],
        out_specs=[row(48), row(48), row(32)],
        out_shape=[jax.ShapeDtypeStruct((NPAD, 48), jnp.float32),
                   jax.ShapeDtypeStruct((NPAD, 48), jnp.float32),
                   jax.ShapeDtypeStruct((NPAD, 32), jnp.float32)],
    )(x, W1f, s0v, b1f, Wl0, bl0, Wd0, c0, Ws0)


def _mid(h, zs, Wd, c, Ws, Wlin, blin, din, dout):
    BM = 512

    def body(hr, zr_in, wdr, cdr, wsr, wlr, blr, ar, br, zr):
        hh = hr[...]
        hq = hh.astype(jnp.bfloat16).astype(jnp.float32)
        ar[...] = jnp.dot(hq, wdr[...], preferred_element_type=jnp.float32, precision=lax.Precision.HIGHEST) + cdr[...]
        br[...] = jnp.dot(hq, wsr[...], preferred_element_type=jnp.float32, precision=lax.Precision.HIGHEST)
        zr[...] = zr_in[...] + jnp.dot(hh, wlr[...], preferred_element_type=jnp.float32) + blr[...]

    row = lambda d: pl.BlockSpec((BM, d), lambda i: (i, 0))
    return pl.pallas_call(
        body,
        grid=(NPAD // BM,),
        in_specs=[row(din), row(32), _full((din, dout)), _full((1, dout)),
                  _full((din, dout)), _full((din, 32)), _full((1, 32))],
        out_specs=[row(dout), row(dout), row(32)],
        out_shape=[jax.ShapeDtypeStruct((NPAD, dout), jnp.float32),
                   jax.ShapeDtypeStruct((NPAD, dout), jnp.float32),
                   jax.ShapeDtypeStruct((NPAD, 32), jnp.float32)],
    )(h, zs, Wd, c, Ws, Wlin, blin)


def _final(zs, h3, batch2d, x_omic, fp):
    def body(zsr, h3r, batr, xor, lwr, lbr,
             m0w, m0b, m1w, m1b, m2w, m2b, m3w, m3b,
             h1w, h1b, z1w, z1b, ow1, ob1,
             h2w, h2b, z2w, z2b, ow2, ob2,
             e1w, e1b, e2w, e2b, cw, cb, feat, haz):
        zs3 = zsr[...] + jnp.dot(h3r[...], lwr[...],
                                 preferred_element_type=jnp.float32) + lbr[...]
        rowid = lax.broadcasted_iota(jnp.int32, (NPAD, 32), 0)
        zs3 = jnp.where(rowid < N_NODES, zs3, 0.0)
        gi = lax.broadcasted_iota(jnp.int32, (NG, NPAD), 0)
        oh = (batr[...] == gi).astype(jnp.float32)
        sums = jnp.dot(oh, zs3, preferred_element_type=jnp.float32, precision=lax.Precision.HIGHEST)
        cnt = jnp.sum(oh, axis=1, keepdims=True)
        v1 = sums / jnp.maximum(cnt, 1.0)

        ho = xor[...]
        for wr, br_ in ((m0w, m0b), (m1w, m1b), (m2w, m2b), (m3w, m3b)):
            t = jnp.dot(ho, wr[...], preferred_element_type=jnp.float32) + br_[...]
            ho = jnp.where(t > 0, t, jnp.exp(jnp.minimum(t, 0.0)) - 1.0)
        v2 = ho

        def bilinear(va, w3r, br_, vb):
            W3 = w3r[...]
            acc = jnp.zeros((NG, 32), jnp.float32)
            for o in range(32):
                t = jnp.dot(va, W3[o], preferred_element_type=jnp.float32)
                s = jnp.sum(t * vb, axis=1, keepdims=True)
                onec = (lax.broadcasted_iota(jnp.int32, (1, 32), 1) == o
                        ).astype(jnp.float32)
                acc = acc + s * onec
            return acc + br_[...]

        hh1 = jnp.maximum(jnp.dot(v1, h1w[...],
                                  preferred_element_type=jnp.float32) + h1b[...], 0.0)
        z1 = bilinear(v1, z1w, z1b, v2)
        sig1 = 1.0 / (1.0 + jnp.exp(-z1))
        oo1 = jnp.maximum(jnp.dot(sig1 * hh1, ow1[...],
                                  preferred_element_type=jnp.float32) + ob1[...], 0.0)
        hh2 = jnp.maximum(jnp.dot(v2, h2w[...],
                                  preferred_element_type=jnp.float32) + h2b[...], 0.0)
        z2 = bilinear(v1, z2w, z2b, v2)
        sig2 = 1.0 / (1.0 + jnp.exp(-z2))
        oo2 = jnp.maximum(jnp.dot(sig2 * hh2, ow2[...],
                                  preferred_element_type=jnp.float32) + ob2[...], 0.0)

        ones = jnp.ones((NG, 1), jnp.float32)
        o1c = jnp.concatenate([oo1, ones], axis=1)
        o2c = jnp.concatenate([oo2, ones], axis=1)
        o12 = jnp.concatenate([o1c[:, i:i + 1] * o2c for i in range(33)],
                              axis=1)
        out1 = jnp.maximum(
            jnp.dot(o12, e1w[...], preferred_element_type=jnp.float32)
            + e1b[...], 0.0)

        E2 = e2w[...]
        pre = (jnp.dot(out1, E2[0:64], preferred_element_type=jnp.float32)
               + jnp.dot(o1c, E2[64:97], preferred_element_type=jnp.float32)
               + jnp.dot(o2c, E2[97:130], preferred_element_type=jnp.float32)
               + e2b[...])
        feats = jnp.maximum(pre, 0.0)
        feat[...] = feats
        haz[...] = jnp.dot(feats, cw[...], preferred_element_type=jnp.float32) + cb[...]

    p = fp
    ins = [zs, h3, batch2d, x_omic, p["lw"], p["lb"],
           p["m0w"], p["m0b"], p["m1w"], p["m1b"], p["m2w"], p["m2b"],
           p["m3w"], p["m3b"],
           p["h1w"], p["h1b"], p["z1w"], p["z1b"], p["ow1"], p["ob1"],
           p["h2w"], p["h2b"], p["z2w"], p["z2b"], p["ow2"], p["ob2"],
           p["e1w"], p["e1b"], p["e2w"], p["e2b"], p["cw"], p["cb"]]
    return pl.pallas_call(
        body,
        in_specs=[_full(v.shape) for v in ins],
        out_specs=[_full((NG, 64)), _full((NG, 1))],
        out_shape=[jax.ShapeDtypeStruct((NG, 64), jnp.float32),
                   jax.ShapeDtypeStruct((NG, 1), jnp.float32)],
    )(*ins)


def kernel(x, edge_index, batch, x_omic, params):
    p = params
    s0 = p["bn0_g"] * _BN
    W1f = p["first_h"]["W"]
    s0v = s0[None, :]
    b1f = (p["first_h"]["b"] * s0 + p["bn0_b"])[None, :]
    Wl0 = p["lin0"]["W"]
    bl0 = p["lin0"]["b"][None, :]

    Wd, Cc, Ws = [], [], []
    dims = [(64, 48), (48, 32), (32, 32)]
    for l, (din, dout) in enumerate(dims):
        cp = p["convs"][l]
        W = cp["lin"]["W"]
        Wt = W[:din].astype(jnp.bfloat16).astype(jnp.float32)
        Wb = W[din:].astype(jnp.bfloat16).astype(jnp.float32)
        s = cp["bn_g"] * _BN
        Wd.append((Wt - Wb) * s[None, :])
        Ws.append(Wb * s[None, :])
        Cc.append((cp["lin"]["b"] * s + cp["bn_b"])[None, :])

    A0, B0, Zs = _front(x, W1f, s0v, b1f, Wl0, bl0, Wd[0], Cc[0], Ws[0])
    csrc, cdl, hlen, hstart = _bin_edges(edge_index)

    h1 = _seg_max(csrc, cdl, hlen, hstart, B0, A0, 48)
    A1, B1, Zs = _mid(h1, Zs, Wd[1], Cc[1], Ws[1],
                      p["lins"][0]["W"], p["lins"][0]["b"][None, :], 48, 32)
    h2 = _seg_max(csrc, cdl, hlen, hstart, B1, A1, 32)
    A2, B2, Zs = _mid(h2, Zs, Wd[2], Cc[2], Ws[2],
                      p["lins"][1]["W"], p["lins"][1]["b"][None, :], 32, 32)
    h3 = _seg_max(csrc, cdl, hlen, hstart, B2, A2, 32)

    batch2d = jnp.concatenate(
        [batch, jnp.full((NPAD - N_NODES,), NG, jnp.int32)]).reshape(1, NPAD)
    fp = {
        "lw": p["lins"][2]["W"], "lb": p["lins"][2]["b"][None, :],
        "m0w": p["omic"][0]["W"], "m0b": p["omic"][0]["b"][None, :],
        "m1w": p["omic"][1]["W"], "m1b": p["omic"][1]["b"][None, :],
        "m2w": p["omic"][2]["W"], "m2b": p["omic"][2]["b"][None, :],
        "m3w": p["omic"][3]["W"], "m3b": p["omic"][3]["b"][None, :],
        "h1w": p["h1"]["W"], "h1b": p["h1"]["b"][None, :],
        "z1w": p["z1_W"], "z1b": p["z1_b"][None, :],
        "ow1": p["o1"]["W"], "ob1": p["o1"]["b"][None, :],
        "h2w": p["h2"]["W"], "h2b": p["h2"]["b"][None, :],
        "z2w": p["z2_W"], "z2b": p["z2_b"][None, :],
        "ow2": p["o2"]["W"], "ob2": p["o2"]["b"][None, :],
        "e1w": p["enc1"]["W"], "e1b": p["enc1"]["b"][None, :],
        "e2w": p["enc2"]["W"], "e2b": p["enc2"]["b"][None, :],
        "cw": p["clf"]["W"], "cb": p["clf"]["b"][None, :],
    }
    features, hazard = _final(Zs, h3, batch2d, x_omic, fp)
    return features, hazard
